# Initial kernel scaffold; baseline (speedup 1.0000x reference)
import jax
import jax.numpy as jnp
from jax.experimental import pallas as pl


def kernel(features, edge_index, table_user, table_item, W1, b1, W2, b2, P1, pb1, P2, pb2):
    raise NotImplementedError("write your pallas kernel here")


# speedup vs baseline: 11.1676x; 11.1676x over previous
import functools

import jax
import jax.numpy as jnp
from jax import lax
from jax.experimental import pallas as pl
from jax.experimental.pallas import tpu as pltpu
from jax.experimental.pallas import tpu_sc as plsc

B = 16384
E = 524288
N = 2 * B
D = 64
DH = D // 2
H = 128

NC = 2
NS = 16
NW = NC * NS
CHUNK = 128
RPT = N // NS


def _mesh():
    return plsc.VectorSubcoreMesh(
        core_axis_name="c", subcore_axis_name="s",
        num_cores=NC, num_subcores=NS)


_SC_PARAMS = pltpu.CompilerParams(use_tc_tiling_on_sc=False)


def _embed_deg_body(tu, ti, f0, f1, dst, zrow,
                    h_out, deg0, deg1,
                    idx_v, rows_v, ones_v, dacc, sem):
    cid = lax.axis_index("c")
    sid = lax.axis_index("s")
    wid = sid * NC + cid

    pltpu.sync_copy(zrow, dacc.at[pl.ds(sid * RPT, RPT)])
    for i in range(CHUNK // 16):
        ones_v[pl.ds(i * 16, 16)] = jnp.ones((16,), jnp.float32)
    plsc.subcore_barrier()

    bp = B // NW

    def gather_table(table, fidx, out_base):
        def body(k, carry):
            base = wid * bp + k * CHUNK
            pltpu.sync_copy(fidx.at[pl.ds(base, CHUNK)], idx_v)
            pltpu.async_copy(table.at[idx_v], rows_v, sem).wait()
            pltpu.sync_copy(rows_v, h_out.at[pl.ds(out_base + base, CHUNK)])
            return carry
        lax.fori_loop(0, bp // CHUNK, body, 0)

    gather_table(tu, f0, 0)
    gather_table(ti, f1, B)

    ep = E // NW

    def dbody(k, carry):
        base = wid * ep + k * CHUNK
        pltpu.sync_copy(dst.at[pl.ds(base, CHUNK)], idx_v)
        pltpu.sync_copy(ones_v, dacc.at[idx_v], add=True)
        return carry
    lax.fori_loop(0, ep // CHUNK, dbody, 0)
    plsc.subcore_barrier()

    @pl.when(cid == 0)
    def _():
        pltpu.sync_copy(dacc.at[pl.ds(sid * RPT, RPT)],
                        deg0.at[pl.ds(sid * RPT, RPT)])

    @pl.when(cid == 1)
    def _():
        pltpu.sync_copy(dacc.at[pl.ds(sid * RPT, RPT)],
                        deg1.at[pl.ds(sid * RPT, RPT)])


_embed_deg = functools.partial(
    pl.kernel,
    out_type=(jax.ShapeDtypeStruct((N, D), jnp.float32),
              jax.ShapeDtypeStruct((N,), jnp.float32),
              jax.ShapeDtypeStruct((N,), jnp.float32)),
    mesh=_mesh(),
    compiler_params=_SC_PARAMS,
    scratch_types=[
        pltpu.VMEM((CHUNK,), jnp.int32),
        pltpu.VMEM((CHUNK, D), jnp.float32),
        pltpu.VMEM((CHUNK,), jnp.float32),
        pltpu.VMEM_SHARED((N,), jnp.float32),
        pltpu.SemaphoreType.DMA,
    ],
)(_embed_deg_body)


def _agg_body(xlo, xhi, src, dst, zblk,
              ylo, yhi,
              sidx, didx, rows, acc, sem):
    cid = lax.axis_index("c")
    sid = lax.axis_index("s")
    ep = E // NS

    def run(x, y):
        pltpu.sync_copy(zblk, acc.at[pl.ds(sid * RPT, RPT)])
        plsc.subcore_barrier()

        def body(k, carry):
            base = sid * ep + k * CHUNK
            pltpu.sync_copy(src.at[pl.ds(base, CHUNK)], sidx)
            pltpu.sync_copy(dst.at[pl.ds(base, CHUNK)], didx)
            pltpu.async_copy(x.at[sidx], rows, sem).wait()
            pltpu.sync_copy(rows, acc.at[didx], add=True)
            return carry
        lax.fori_loop(0, ep // CHUNK, body, 0)
        plsc.subcore_barrier()
        pltpu.sync_copy(acc.at[pl.ds(sid * RPT, RPT)],
                        y.at[pl.ds(sid * RPT, RPT)])

    @pl.when(cid == 0)
    def _():
        run(xlo, ylo)

    @pl.when(cid == 1)
    def _():
        run(xhi, yhi)


_agg = functools.partial(
    pl.kernel,
    out_type=(jax.ShapeDtypeStruct((N, DH), jnp.float32),
              jax.ShapeDtypeStruct((N, DH), jnp.float32)),
    mesh=_mesh(),
    compiler_params=_SC_PARAMS,
    scratch_types=[
        pltpu.VMEM((CHUNK,), jnp.int32),
        pltpu.VMEM((CHUNK,), jnp.int32),
        pltpu.VMEM((CHUNK, DH), jnp.float32),
        pltpu.VMEM_SHARED((N, DH), jnp.float32),
        pltpu.SemaphoreType.DMA,
    ],
)(_agg_body)


RT = 2048


def _scale_body(h_ref, d0_ref, d1_ref, hlo_ref, hhi_ref, r_ref):
    d = d0_ref[...] + d1_ref[...]
    r = lax.rsqrt(jnp.maximum(d, 1.0))
    hs = h_ref[...] * r
    hlo_ref[...] = hs[:, :DH]
    hhi_ref[...] = hs[:, DH:]
    r_ref[...] = r


def _scale(h, d0, d1):
    return pl.pallas_call(
        _scale_body,
        grid=(N // RT,),
        in_specs=[
            pl.BlockSpec((RT, D), lambda i: (i, 0)),
            pl.BlockSpec((RT, 1), lambda i: (i, 0)),
            pl.BlockSpec((RT, 1), lambda i: (i, 0)),
        ],
        out_specs=[
            pl.BlockSpec((RT, DH), lambda i: (i, 0)),
            pl.BlockSpec((RT, DH), lambda i: (i, 0)),
            pl.BlockSpec((RT, 1), lambda i: (i, 0)),
        ],
        out_shape=[
            jax.ShapeDtypeStruct((N, DH), jnp.float32),
            jax.ShapeDtypeStruct((N, DH), jnp.float32),
            jax.ShapeDtypeStruct((N, 1), jnp.float32),
        ],
    )(h, d0, d1)


def _mid_body(ylo, yhi, r_ref, w1_ref, b1_ref, w2_ref, glo, ghi):
    r = r_ref[...]
    agg = jnp.concatenate([ylo[...], yhi[...]], axis=1) * r
    h1 = jnp.maximum(
        jnp.dot(agg, w1_ref[...], preferred_element_type=jnp.float32)
        + b1_ref[...], 0.0)
    gs = jnp.dot(h1, w2_ref[...], preferred_element_type=jnp.float32) * r
    glo[...] = gs[:, :DH]
    ghi[...] = gs[:, DH:]


def _mid(ylo, yhi, r, w1, b1, w2):
    return pl.pallas_call(
        _mid_body,
        grid=(N // RT,),
        in_specs=[
            pl.BlockSpec((RT, DH), lambda i: (i, 0)),
            pl.BlockSpec((RT, DH), lambda i: (i, 0)),
            pl.BlockSpec((RT, 1), lambda i: (i, 0)),
            pl.BlockSpec((D, H), lambda i: (0, 0)),
            pl.BlockSpec((1, H), lambda i: (0, 0)),
            pl.BlockSpec((H, D), lambda i: (0, 0)),
        ],
        out_specs=[
            pl.BlockSpec((RT, DH), lambda i: (i, 0)),
            pl.BlockSpec((RT, DH), lambda i: (i, 0)),
        ],
        out_shape=[
            jax.ShapeDtypeStruct((N, DH), jnp.float32),
            jax.ShapeDtypeStruct((N, DH), jnp.float32),
        ],
    )(ylo, yhi, r, w1, b1, w2)


def _final_body(zlo_u, zhi_u, zlo_i, zhi_i, r_u, r_i,
                b2_ref, p1_ref, pb1_ref, p2_ref, pb2_ref, o_ref):
    h2u = (jnp.concatenate([zlo_u[...], zhi_u[...]], axis=1) * r_u[...]
           + b2_ref[...])
    h2i = (jnp.concatenate([zlo_i[...], zhi_i[...]], axis=1) * r_i[...]
           + b2_ref[...])
    x = jnp.concatenate([h2u, h2i], axis=1)
    z = jnp.maximum(
        jnp.dot(x, p1_ref[...], preferred_element_type=jnp.float32)
        + pb1_ref[...], 0.0)
    o_ref[...] = (jnp.dot(z, p2_ref[...], preferred_element_type=jnp.float32)
                  + pb2_ref[...])


def _final(zlo, zhi, r, b2, p1, pb1, p2, pb2):
    ioff = B // RT
    return pl.pallas_call(
        _final_body,
        grid=(B // RT,),
        in_specs=[
            pl.BlockSpec((RT, DH), lambda i: (i, 0)),
            pl.BlockSpec((RT, DH), lambda i: (i, 0)),
            pl.BlockSpec((RT, DH), lambda i: (i + ioff, 0)),
            pl.BlockSpec((RT, DH), lambda i: (i + ioff, 0)),
            pl.BlockSpec((RT, 1), lambda i: (i, 0)),
            pl.BlockSpec((RT, 1), lambda i: (i + ioff, 0)),
            pl.BlockSpec((1, D), lambda i: (0, 0)),
            pl.BlockSpec((H, H), lambda i: (0, 0)),
            pl.BlockSpec((1, H), lambda i: (0, 0)),
            pl.BlockSpec((H, 1), lambda i: (0, 0)),
            pl.BlockSpec((1, 1), lambda i: (0, 0)),
        ],
        out_specs=pl.BlockSpec((RT, 1), lambda i: (i, 0)),
        out_shape=jax.ShapeDtypeStruct((B, 1), jnp.float32),
    )(zlo, zhi, zlo, zhi, r, r, b2, p1, pb1, p2, pb2)


def kernel(features, edge_index, table_user, table_item,
           W1, b1, W2, b2, P1, pb1, P2, pb2):
    f0 = features[0].astype(jnp.int32)
    f1 = features[1].astype(jnp.int32)
    src = edge_index[0].astype(jnp.int32)
    dst = edge_index[1].astype(jnp.int32)
    zrow = jnp.zeros((RPT,), jnp.float32)
    zblk = jnp.zeros((RPT, DH), jnp.float32)

    h, deg0, deg1 = _embed_deg(table_user, table_item, f0, f1, dst, zrow)
    hs_lo, hs_hi, r = _scale(h, deg0.reshape(N, 1), deg1.reshape(N, 1))
    y_lo, y_hi = _agg(hs_lo, hs_hi, src, dst, zblk)
    gs_lo, gs_hi = _mid(y_lo, y_hi, r, W1, b1.reshape(1, H), W2)
    z_lo, z_hi = _agg(gs_lo, gs_hi, src, dst, zblk)
    return _final(z_lo, z_hi, r, b2.reshape(1, D), P1,
                  pb1.reshape(1, H), P2, pb2.reshape(1, 1))


# speedup vs baseline: 25.2999x; 2.2655x over previous
import functools

import jax
import jax.numpy as jnp
from jax import lax
from jax.experimental import pallas as pl
from jax.experimental.pallas import tpu as pltpu
from jax.experimental.pallas import tpu_sc as plsc

B = 16384
E = 524288
N = 2 * B
D = 64
DH = D // 2
H = 128

NC = 2
NS = 16
NW = NC * NS
CHUNK = 128
RPT = N // NS


def _mesh():
    return plsc.VectorSubcoreMesh(
        core_axis_name="c", subcore_axis_name="s",
        num_cores=NC, num_subcores=NS)


_SC_PARAMS = pltpu.CompilerParams(use_tc_tiling_on_sc=False)


def _embed_deg_body(tu, ti, f0, f1, dst, zrow,
                    h_out, deg0, deg1,
                    idx_v, rows_v, ones_v, dacc, sem):
    cid = lax.axis_index("c")
    sid = lax.axis_index("s")
    wid = sid * NC + cid

    pltpu.sync_copy(zrow, dacc.at[pl.ds(sid * RPT, RPT)])
    for i in range(CHUNK // 16):
        ones_v[pl.ds(i * 16, 16)] = jnp.ones((16,), jnp.float32)
    plsc.subcore_barrier()

    bp = B // NW

    def gather_table(table, fidx, out_base):
        def body(k, carry):
            base = wid * bp + k * CHUNK
            pltpu.sync_copy(fidx.at[pl.ds(base, CHUNK)], idx_v)
            pltpu.async_copy(table.at[idx_v], rows_v, sem).wait()
            pltpu.sync_copy(rows_v, h_out.at[pl.ds(out_base + base, CHUNK)])
            return carry
        lax.fori_loop(0, bp // CHUNK, body, 0)

    gather_table(tu, f0, 0)
    gather_table(ti, f1, B)

    ep = E // NW

    def dbody(k, carry):
        base = wid * ep + k * CHUNK
        pltpu.sync_copy(dst.at[pl.ds(base, CHUNK)], idx_v)
        pltpu.sync_copy(ones_v, dacc.at[idx_v], add=True)
        return carry
    lax.fori_loop(0, ep // CHUNK, dbody, 0)
    plsc.subcore_barrier()

    @pl.when(cid == 0)
    def _():
        pltpu.sync_copy(dacc.at[pl.ds(sid * RPT, RPT)],
                        deg0.at[pl.ds(sid * RPT, RPT)])

    @pl.when(cid == 1)
    def _():
        pltpu.sync_copy(dacc.at[pl.ds(sid * RPT, RPT)],
                        deg1.at[pl.ds(sid * RPT, RPT)])


_embed_deg = functools.partial(
    pl.kernel,
    out_type=(jax.ShapeDtypeStruct((N, D), jnp.float32),
              jax.ShapeDtypeStruct((N,), jnp.float32),
              jax.ShapeDtypeStruct((N,), jnp.float32)),
    mesh=_mesh(),
    compiler_params=_SC_PARAMS,
    scratch_types=[
        pltpu.VMEM((CHUNK,), jnp.int32),
        pltpu.VMEM((CHUNK, D), jnp.float32),
        pltpu.VMEM((CHUNK,), jnp.float32),
        pltpu.VMEM_SHARED((N,), jnp.float32),
        pltpu.SemaphoreType.DMA,
    ],
)(_embed_deg_body)


GRP = 4
NB = 2 * GRP
CPT = (E // NS) // CHUNK
SUPB = 64
NSUP = CPT // SUPB
NGRP = SUPB // GRP
NITER = NGRP // 2 - 1


def _agg_body(xlo, xhi, src2, dst2, zblk,
              ylo, yhi,
              sidx, didx, rows, acc, *sems):
    gsem = sems[:NB]
    ssem = sems[NB:]
    cid = lax.axis_index("c")
    sid = lax.axis_index("s")

    def run(x, y):
        pltpu.sync_copy(zblk, acc.at[pl.ds(sid * RPT, RPT)])
        plsc.subcore_barrier()

        def fire_gather(b, j):
            pltpu.async_copy(x.at[sidx.at[j]], rows.at[b], gsem[b])

        def wait_gather(b, j):
            pltpu.make_async_copy(x.at[sidx.at[j]], rows.at[b],
                                  gsem[b]).wait()

        def fire_scatter(b, j):
            pltpu.async_copy(rows.at[b], acc.at[didx.at[j]], ssem[b],
                             add=True)

        def wait_scatter(b, j):
            pltpu.make_async_copy(rows.at[b], acc.at[didx.at[j]],
                                  ssem[b]).wait()

        def sup_body(sb, carry):
            row0 = sid * CPT + sb * SUPB
            pltpu.sync_copy(src2.at[pl.ds(row0, SUPB)], sidx)
            pltpu.sync_copy(dst2.at[pl.ds(row0, SUPB)], didx)

            for b in range(NB):
                fire_gather(b, b)

            def body(gg, c):
                j0 = 2 * gg * GRP
                for b in range(GRP):
                    wait_gather(b, j0 + b)
                    fire_scatter(b, j0 + b)
                for b in range(GRP):
                    wait_gather(GRP + b, j0 + GRP + b)
                    fire_scatter(GRP + b, j0 + GRP + b)
                for b in range(GRP):
                    wait_scatter(b, j0 + b)
                    fire_gather(b, j0 + 2 * GRP + b)
                for b in range(GRP):
                    wait_scatter(GRP + b, j0 + GRP + b)
                    fire_gather(GRP + b, j0 + 3 * GRP + b)
                return c
            lax.fori_loop(0, NITER, body, 0)

            je = (NGRP - 2) * GRP
            for b in range(NB):
                wait_gather(b, je + b)
                fire_scatter(b, je + b)
            for b in range(NB):
                wait_scatter(b, je + b)
            return carry
        lax.fori_loop(0, NSUP, sup_body, 0)
        plsc.subcore_barrier()
        pltpu.sync_copy(acc.at[pl.ds(sid * RPT, RPT)],
                        y.at[pl.ds(sid * RPT, RPT)])

    @pl.when(cid == 0)
    def _():
        run(xlo, ylo)

    @pl.when(cid == 1)
    def _():
        run(xhi, yhi)


_agg = functools.partial(
    pl.kernel,
    out_type=(jax.ShapeDtypeStruct((N, DH), jnp.float32),
              jax.ShapeDtypeStruct((N, DH), jnp.float32)),
    mesh=_mesh(),
    compiler_params=_SC_PARAMS,
    scratch_types=[
        pltpu.VMEM((SUPB, CHUNK), jnp.int32),
        pltpu.VMEM((SUPB, CHUNK), jnp.int32),
        pltpu.VMEM((NB, CHUNK, DH), jnp.float32),
        pltpu.VMEM_SHARED((N, DH), jnp.float32),
    ] + [pltpu.SemaphoreType.DMA] * (2 * NB),
)(_agg_body)


RT = 2048


def _scale_body(h_ref, d0_ref, d1_ref, hlo_ref, hhi_ref, r_ref):
    d = d0_ref[...] + d1_ref[...]
    r = lax.rsqrt(jnp.maximum(d, 1.0))
    hs = h_ref[...] * r
    hlo_ref[...] = hs[:, :DH]
    hhi_ref[...] = hs[:, DH:]
    r_ref[...] = r


def _scale(h, d0, d1):
    return pl.pallas_call(
        _scale_body,
        grid=(N // RT,),
        in_specs=[
            pl.BlockSpec((RT, D), lambda i: (i, 0)),
            pl.BlockSpec((RT, 1), lambda i: (i, 0)),
            pl.BlockSpec((RT, 1), lambda i: (i, 0)),
        ],
        out_specs=[
            pl.BlockSpec((RT, DH), lambda i: (i, 0)),
            pl.BlockSpec((RT, DH), lambda i: (i, 0)),
            pl.BlockSpec((RT, 1), lambda i: (i, 0)),
        ],
        out_shape=[
            jax.ShapeDtypeStruct((N, DH), jnp.float32),
            jax.ShapeDtypeStruct((N, DH), jnp.float32),
            jax.ShapeDtypeStruct((N, 1), jnp.float32),
        ],
    )(h, d0, d1)


def _mid_body(ylo, yhi, r_ref, w1_ref, b1_ref, w2_ref, glo, ghi):
    r = r_ref[...]
    agg = jnp.concatenate([ylo[...], yhi[...]], axis=1) * r
    h1 = jnp.maximum(
        jnp.dot(agg, w1_ref[...], preferred_element_type=jnp.float32)
        + b1_ref[...], 0.0)
    gs = jnp.dot(h1, w2_ref[...], preferred_element_type=jnp.float32) * r
    glo[...] = gs[:, :DH]
    ghi[...] = gs[:, DH:]


def _mid(ylo, yhi, r, w1, b1, w2):
    return pl.pallas_call(
        _mid_body,
        grid=(N // RT,),
        in_specs=[
            pl.BlockSpec((RT, DH), lambda i: (i, 0)),
            pl.BlockSpec((RT, DH), lambda i: (i, 0)),
            pl.BlockSpec((RT, 1), lambda i: (i, 0)),
            pl.BlockSpec((D, H), lambda i: (0, 0)),
            pl.BlockSpec((1, H), lambda i: (0, 0)),
            pl.BlockSpec((H, D), lambda i: (0, 0)),
        ],
        out_specs=[
            pl.BlockSpec((RT, DH), lambda i: (i, 0)),
            pl.BlockSpec((RT, DH), lambda i: (i, 0)),
        ],
        out_shape=[
            jax.ShapeDtypeStruct((N, DH), jnp.float32),
            jax.ShapeDtypeStruct((N, DH), jnp.float32),
        ],
    )(ylo, yhi, r, w1, b1, w2)


def _final_body(zlo_u, zhi_u, zlo_i, zhi_i, r_u, r_i,
                b2_ref, p1_ref, pb1_ref, p2_ref, pb2_ref, o_ref):
    h2u = (jnp.concatenate([zlo_u[...], zhi_u[...]], axis=1) * r_u[...]
           + b2_ref[...])
    h2i = (jnp.concatenate([zlo_i[...], zhi_i[...]], axis=1) * r_i[...]
           + b2_ref[...])
    x = jnp.concatenate([h2u, h2i], axis=1)
    z = jnp.maximum(
        jnp.dot(x, p1_ref[...], preferred_element_type=jnp.float32)
        + pb1_ref[...], 0.0)
    o_ref[...] = (jnp.dot(z, p2_ref[...], preferred_element_type=jnp.float32)
                  + pb2_ref[...])


def _final(zlo, zhi, r, b2, p1, pb1, p2, pb2):
    ioff = B // RT
    return pl.pallas_call(
        _final_body,
        grid=(B // RT,),
        in_specs=[
            pl.BlockSpec((RT, DH), lambda i: (i, 0)),
            pl.BlockSpec((RT, DH), lambda i: (i, 0)),
            pl.BlockSpec((RT, DH), lambda i: (i + ioff, 0)),
            pl.BlockSpec((RT, DH), lambda i: (i + ioff, 0)),
            pl.BlockSpec((RT, 1), lambda i: (i, 0)),
            pl.BlockSpec((RT, 1), lambda i: (i + ioff, 0)),
            pl.BlockSpec((1, D), lambda i: (0, 0)),
            pl.BlockSpec((H, H), lambda i: (0, 0)),
            pl.BlockSpec((1, H), lambda i: (0, 0)),
            pl.BlockSpec((H, 1), lambda i: (0, 0)),
            pl.BlockSpec((1, 1), lambda i: (0, 0)),
        ],
        out_specs=pl.BlockSpec((RT, 1), lambda i: (i, 0)),
        out_shape=jax.ShapeDtypeStruct((B, 1), jnp.float32),
    )(zlo, zhi, zlo, zhi, r, r, b2, p1, pb1, p2, pb2)


def kernel(features, edge_index, table_user, table_item,
           W1, b1, W2, b2, P1, pb1, P2, pb2):
    f0 = features[0].astype(jnp.int32)
    f1 = features[1].astype(jnp.int32)
    src = edge_index[0].astype(jnp.int32)
    dst = edge_index[1].astype(jnp.int32)
    zrow = jnp.zeros((RPT,), jnp.float32)
    zblk = jnp.zeros((RPT, DH), jnp.float32)

    src2 = src.reshape(E // CHUNK, CHUNK)
    dst2 = dst.reshape(E // CHUNK, CHUNK)

    h, deg0, deg1 = _embed_deg(table_user, table_item, f0, f1, dst, zrow)
    hs_lo, hs_hi, r = _scale(h, deg0.reshape(N, 1), deg1.reshape(N, 1))
    y_lo, y_hi = _agg(hs_lo, hs_hi, src2, dst2, zblk)
    gs_lo, gs_hi = _mid(y_lo, y_hi, r, W1, b1.reshape(1, H), W2)
    z_lo, z_hi = _agg(gs_lo, gs_hi, src2, dst2, zblk)
    return _final(z_lo, z_hi, r, b2.reshape(1, D), P1,
                  pb1.reshape(1, H), P2, pb2.reshape(1, 1))
